# baseline (device time: 17494 ns/iter reference)
import jax
import jax.numpy as jnp
from jax import lax
from jax.experimental import pallas as pl
from jax.experimental.pallas import tpu as pltpu

N_DEV = 4
B, SQ, SKV, D_MODEL = 2, 128, 128, 512
HQ_LOCAL, DH = 4, 64
BLK = 64


def _body(x_ref, wq_ref, k_any, v_any, wo_ref, out_ref,
          kv_ref, send_ref, recv_ref, kv_sems, send_sems, recv_sems):
    my_pos = lax.axis_index("i")
    p_diag = my_pos ^ 2
    p_ring = my_pos ^ 1

    barrier_sem = pltpu.get_barrier_semaphore()
    for p in (p_diag, p_ring):
        pl.semaphore_signal(
            barrier_sem, inc=1,
            device_id=(p,), device_id_type=pl.DeviceIdType.MESH,
        )

    kv_dmas = []
    for kv, src in ((0, k_any), (1, v_any)):
        for b in range(B):
            for h in range(HQ_LOCAL):
                c = pltpu.make_async_copy(
                    src.at[b, :, my_pos * HQ_LOCAL + h, :],
                    kv_ref.at[kv, b, h],
                    kv_sems.at[kv, b, h],
                )
                c.start()
                kv_dmas.append(c)

    wq = wq_ref[:].astype(jnp.bfloat16)
    wo = wo_ref[:].astype(jnp.bfloat16)

    def partial_for_batch(b):
        q = lax.dot_general(
            x_ref[b].astype(jnp.bfloat16), wq, (((1,), (0,)), ((), ())),
            preferred_element_type=jnp.float32,
        )
        q = (q * 0.125).astype(jnp.bfloat16)
        if b == 0:
            for c in kv_dmas:
                c.wait()
        pacc = None
        for h in range(HQ_LOCAL):
            k = kv_ref[0, b, h].astype(jnp.bfloat16)
            v = kv_ref[1, b, h].astype(jnp.bfloat16)
            blocks = []
            for blk in range(2):
                rows = slice(blk * BLK, (blk + 1) * BLK)
                s = lax.dot_general(
                    q[rows, h * DH:(h + 1) * DH], k[rows],
                    (((1,), (1,)), ((), ())),
                    preferred_element_type=jnp.float32,
                )
                w = jnp.exp(s)
                r = 1.0 / jnp.sum(w, axis=-1, keepdims=True)
                ctx = lax.dot_general(
                    w.astype(jnp.bfloat16), v[rows], (((1,), (0,)), ((), ())),
                    preferred_element_type=jnp.float32,
                )
                blocks.append((ctx * r).astype(jnp.bfloat16))
            ctx_h = jnp.concatenate(blocks, axis=0)
            p_h = lax.dot_general(
                ctx_h, wo[h * DH:(h + 1) * DH], (((1,), (0,)), ((), ())),
                preferred_element_type=jnp.float32,
            )
            pacc = p_h if pacc is None else pacc + p_h
        return pacc

    def exchange(slot, peer):
        return pltpu.make_async_remote_copy(
            src_ref=send_ref.at[slot],
            dst_ref=recv_ref.at[slot],
            send_sem=send_sems.at[slot],
            recv_sem=recv_sems.at[slot],
            device_id=(peer,),
            device_id_type=pl.DeviceIdType.MESH,
        )

    partials, r1 = [], []
    for b in range(B):
        pb = partial_for_batch(b)
        partials.append(pb)
        send_ref[b] = pb.astype(jnp.bfloat16)
        if b == 0:
            pl.semaphore_wait(barrier_sem, 2)
        r1.append(exchange(b, p_diag))
        r1[b].start()

    accs, r2 = [], []
    for b in range(B):
        r1[b].wait_recv()
        acc = partials[b] + recv_ref[b].astype(jnp.float32)
        accs.append(acc)
        send_ref[B + b] = acc.astype(jnp.bfloat16)
        r2.append(exchange(B + b, p_ring))
        r2[b].start()

    for b in range(B):
        r2[b].wait_recv()
        out_ref[b] = accs[b] + recv_ref[B + b].astype(jnp.float32)

    for b in range(B):
        r1[b].wait_send()
        r2[b].wait_send()


def kernel(x, Wq, K_ext, V_ext, Wo):
    return pl.pallas_call(
        _body,
        out_shape=jax.ShapeDtypeStruct((B, SQ, D_MODEL), jnp.float32),
        in_specs=[
            pl.BlockSpec(memory_space=pltpu.VMEM),
            pl.BlockSpec(memory_space=pltpu.VMEM),
            pl.BlockSpec(memory_space=pltpu.MemorySpace.HBM),
            pl.BlockSpec(memory_space=pltpu.MemorySpace.HBM),
            pl.BlockSpec(memory_space=pltpu.VMEM),
        ],
        out_specs=pl.BlockSpec(memory_space=pltpu.VMEM),
        scratch_shapes=[
            pltpu.VMEM((2, B, HQ_LOCAL, SKV, DH), jnp.float32),
            pltpu.VMEM((2 * B, SQ, D_MODEL), jnp.bfloat16),
            pltpu.VMEM((2 * B, SQ, D_MODEL), jnp.bfloat16),
            pltpu.SemaphoreType.DMA((2, B, HQ_LOCAL)),
            pltpu.SemaphoreType.DMA((2 * B,)),
            pltpu.SemaphoreType.DMA((2 * B,)),
        ],
        compiler_params=pltpu.CompilerParams(collective_id=0),
    )(x, Wq, K_ext, V_ext, Wo)


# device time: 15923 ns/iter; 1.0987x vs baseline; 1.0987x over previous
import jax
import jax.numpy as jnp
from jax import lax
from jax.experimental import pallas as pl
from jax.experimental.pallas import tpu as pltpu

N_DEV = 4
B, SQ, SKV, D_MODEL = 2, 128, 128, 512
HQ_LOCAL, DH = 4, 64
D_LOCAL = HQ_LOCAL * DH
BLK = 64


def _body(x_ref, wq_ref, k_any, v_any, wo_ref, out_ref,
          kv_ref, send_ref, recv_ref, kv_sems, send_sems, recv_sems):
    my_pos = lax.axis_index("i")
    p1 = my_pos ^ 1
    p2 = jnp.where(my_pos % 2 == 1, (my_pos + 1) % N_DEV,
                   (my_pos - 1) % N_DEV)

    barrier_sem = pltpu.get_barrier_semaphore()
    for p in (p1, p2):
        pl.semaphore_signal(
            barrier_sem, inc=1,
            device_id=(p,), device_id_type=pl.DeviceIdType.MESH,
        )

    kv_dmas = []
    for slot, src in ((0, k_any), (1, v_any)):
        c = pltpu.make_async_copy(
            src.at[:, :, pl.ds(my_pos * D_LOCAL, D_LOCAL)],
            kv_ref.at[slot],
            kv_sems.at[slot],
        )
        c.start()
        kv_dmas.append(c)

    wq = wq_ref[:].astype(jnp.bfloat16)
    wo = wo_ref[:].astype(jnp.bfloat16)

    def partial_for_batch(b):
        q = lax.dot_general(
            x_ref[b].astype(jnp.bfloat16), wq, (((1,), (0,)), ((), ())),
            preferred_element_type=jnp.float32,
        )
        q = (q * 0.125).astype(jnp.bfloat16)
        if b == 0:
            for c in kv_dmas:
                c.wait()
        pacc = None
        for h in range(HQ_LOCAL):
            cols = slice(h * DH, (h + 1) * DH)
            blocks = []
            for blk in range(2):
                rows = slice(blk * BLK, (blk + 1) * BLK)
                k = kv_ref[0, b, rows, cols].astype(jnp.bfloat16)
                v = kv_ref[1, b, rows, cols].astype(jnp.bfloat16)
                s = lax.dot_general(
                    q[rows, cols], k, (((1,), (1,)), ((), ())),
                    preferred_element_type=jnp.float32,
                )
                w = jnp.exp(s)
                r = 1.0 / jnp.sum(w, axis=-1, keepdims=True)
                ctx = lax.dot_general(
                    w.astype(jnp.bfloat16), v, (((1,), (0,)), ((), ())),
                    preferred_element_type=jnp.float32,
                )
                blocks.append((ctx * r).astype(jnp.bfloat16))
            ctx_h = jnp.concatenate(blocks, axis=0)
            p_h = lax.dot_general(
                ctx_h, wo[h * DH:(h + 1) * DH], (((1,), (0,)), ((), ())),
                preferred_element_type=jnp.float32,
            )
            pacc = p_h if pacc is None else pacc + p_h
        return pacc

    def exchange(slot, peer):
        return pltpu.make_async_remote_copy(
            src_ref=send_ref.at[slot],
            dst_ref=recv_ref.at[slot],
            send_sem=send_sems.at[slot],
            recv_sem=recv_sems.at[slot],
            device_id=(peer,),
            device_id_type=pl.DeviceIdType.MESH,
        )

    partials, r1 = [], []
    for b in range(B):
        pb = partial_for_batch(b)
        partials.append(pb)
        send_ref[b] = pb.astype(jnp.bfloat16)
        if b == 0:
            pl.semaphore_wait(barrier_sem, 2)
        r1.append(exchange(b, p1))
        r1[b].start()

    accs, r2 = [], []
    for b in range(B):
        r1[b].wait_recv()
        acc = partials[b] + recv_ref[b].astype(jnp.float32)
        accs.append(acc)
        send_ref[B + b] = acc.astype(jnp.bfloat16)
        r2.append(exchange(B + b, p2))
        r2[b].start()

    for b in range(B):
        r2[b].wait_recv()
        out_ref[b] = accs[b] + recv_ref[B + b].astype(jnp.float32)

    for b in range(B):
        r1[b].wait_send()
        r2[b].wait_send()


def kernel(x, Wq, K_ext, V_ext, Wo):
    k_flat = K_ext.reshape(B, SKV, 16 * DH)
    v_flat = V_ext.reshape(B, SKV, 16 * DH)
    return pl.pallas_call(
        _body,
        out_shape=jax.ShapeDtypeStruct((B, SQ, D_MODEL), jnp.float32),
        in_specs=[
            pl.BlockSpec(memory_space=pltpu.VMEM),
            pl.BlockSpec(memory_space=pltpu.VMEM),
            pl.BlockSpec(memory_space=pltpu.MemorySpace.HBM),
            pl.BlockSpec(memory_space=pltpu.MemorySpace.HBM),
            pl.BlockSpec(memory_space=pltpu.VMEM),
        ],
        out_specs=pl.BlockSpec(memory_space=pltpu.VMEM),
        scratch_shapes=[
            pltpu.VMEM((2, B, SKV, D_LOCAL), jnp.float32),
            pltpu.VMEM((2 * B, SQ, D_MODEL), jnp.bfloat16),
            pltpu.VMEM((2 * B, SQ, D_MODEL), jnp.bfloat16),
            pltpu.SemaphoreType.DMA((2,)),
            pltpu.SemaphoreType.DMA((2 * B,)),
            pltpu.SemaphoreType.DMA((2 * B,)),
        ],
        compiler_params=pltpu.CompilerParams(collective_id=0),
    )(x, Wq, k_flat, v_flat, Wo)


# device time: 14182 ns/iter; 1.2335x vs baseline; 1.1228x over previous
import jax
import jax.numpy as jnp
from jax import lax
from jax.experimental import pallas as pl
from jax.experimental.pallas import tpu as pltpu

N_DEV = 4
B, SQ, SKV, D_MODEL = 2, 128, 128, 512
HQ_LOCAL, DH = 4, 64
D_LOCAL = HQ_LOCAL * DH
BLK = 64


def _body(x_ref, wq_ref, k_ref, v_ref, wo_ref, out_ref,
          send_ref, recv_ref, send_sems, recv_sems):
    my_pos = lax.axis_index("i")
    p1 = my_pos ^ 1
    p2 = jnp.where(my_pos % 2 == 1, (my_pos + 1) % N_DEV,
                   (my_pos - 1) % N_DEV)

    barrier_sem = pltpu.get_barrier_semaphore()
    for p in (p1, p2):
        pl.semaphore_signal(
            barrier_sem, inc=1,
            device_id=(p,), device_id_type=pl.DeviceIdType.MESH,
        )

    wq = wq_ref[:].astype(jnp.bfloat16)
    wo = wo_ref[:].astype(jnp.bfloat16)

    def partial_for_batch(b):
        q = lax.dot_general(
            x_ref[b].astype(jnp.bfloat16), wq, (((1,), (0,)), ((), ())),
            preferred_element_type=jnp.float32,
        )
        q = (q * 0.125).astype(jnp.bfloat16)
        pacc = None
        for h in range(HQ_LOCAL):
            cols = slice(h * DH, (h + 1) * DH)
            blocks = []
            for blk in range(2):
                rows = slice(blk * BLK, (blk + 1) * BLK)
                k = k_ref[b, rows, cols].astype(jnp.bfloat16)
                v = v_ref[b, rows, cols].astype(jnp.bfloat16)
                s = lax.dot_general(
                    q[rows, cols], k, (((1,), (1,)), ((), ())),
                    preferred_element_type=jnp.float32,
                )
                w = jnp.exp(s)
                r = 1.0 / jnp.sum(w, axis=-1, keepdims=True)
                ctx = lax.dot_general(
                    w.astype(jnp.bfloat16), v, (((1,), (0,)), ((), ())),
                    preferred_element_type=jnp.float32,
                )
                blocks.append((ctx * r).astype(jnp.bfloat16))
            ctx_h = jnp.concatenate(blocks, axis=0)
            p_h = lax.dot_general(
                ctx_h, wo[h * DH:(h + 1) * DH], (((1,), (0,)), ((), ())),
                preferred_element_type=jnp.float32,
            )
            pacc = p_h if pacc is None else pacc + p_h
        return pacc

    def partial_for_quarter(b, blk):
        rows = slice(blk * BLK, (blk + 1) * BLK)
        q = lax.dot_general(
            x_ref[b, rows].astype(jnp.bfloat16), wq, (((1,), (0,)), ((), ())),
            preferred_element_type=jnp.float32,
        )
        q = (q * 0.125).astype(jnp.bfloat16)
        pacc = None
        for h in range(HQ_LOCAL):
            cols = slice(h * DH, (h + 1) * DH)
            k = k_ref[b, rows, cols].astype(jnp.bfloat16)
            v = v_ref[b, rows, cols].astype(jnp.bfloat16)
            s = lax.dot_general(
                q[:, cols], k, (((1,), (1,)), ((), ())),
                preferred_element_type=jnp.float32,
            )
            w = jnp.exp(s)
            r = 1.0 / jnp.sum(w, axis=-1, keepdims=True)
            ctx = lax.dot_general(
                w.astype(jnp.bfloat16), v, (((1,), (0,)), ((), ())),
                preferred_element_type=jnp.float32,
            )
            p_h = lax.dot_general(
                (ctx * r).astype(jnp.bfloat16), wo[h * DH:(h + 1) * DH],
                (((1,), (0,)), ((), ())),
                preferred_element_type=jnp.float32,
            )
            pacc = p_h if pacc is None else pacc + p_h
        return pacc

    def exchange(slot, peer):
        return pltpu.make_async_remote_copy(
            src_ref=send_ref.at[slot],
            dst_ref=recv_ref.at[slot],
            send_sem=send_sems.at[slot],
            recv_sem=recv_sems.at[slot],
            device_id=(peer,),
            device_id_type=pl.DeviceIdType.MESH,
        )

    NP = 2 * B
    pieces, r1 = [], []
    p0 = partial_for_batch(0)
    for j in range(2):
        pieces.append(p0[j * BLK:(j + 1) * BLK])
        send_ref[j] = pieces[j].astype(jnp.bfloat16)
        if j == 0:
            pl.semaphore_wait(barrier_sem, 2)
        r1.append(exchange(j, p1))
        r1[j].start()
    for j in range(2):
        t = 2 + j
        pieces.append(partial_for_quarter(1, j))
        send_ref[t] = pieces[t].astype(jnp.bfloat16)
        r1.append(exchange(t, p1))
        r1[t].start()

    accs, r2 = [], []
    for t in range(NP):
        r1[t].wait_recv()
        acc = pieces[t] + recv_ref[t].astype(jnp.float32)
        accs.append(acc)
        send_ref[NP + t] = acc.astype(jnp.bfloat16)
        r2.append(exchange(NP + t, p2))
        r2[t].start()

    for t in range(NP):
        b, j = divmod(t, 2)
        r2[t].wait_recv()
        out_ref[b, j * BLK:(j + 1) * BLK] = (
            accs[t] + recv_ref[NP + t].astype(jnp.float32))

    for t in range(NP):
        r1[t].wait_send()
        r2[t].wait_send()


def kernel(x, Wq, K_ext, V_ext, Wo):
    my_pos = lax.axis_index("i")
    k_loc = lax.dynamic_slice_in_dim(
        K_ext, my_pos * HQ_LOCAL, HQ_LOCAL, axis=2).reshape(B, SKV, D_LOCAL)
    v_loc = lax.dynamic_slice_in_dim(
        V_ext, my_pos * HQ_LOCAL, HQ_LOCAL, axis=2).reshape(B, SKV, D_LOCAL)
    return pl.pallas_call(
        _body,
        out_shape=jax.ShapeDtypeStruct((B, SQ, D_MODEL), jnp.float32),
        in_specs=[pl.BlockSpec(memory_space=pltpu.VMEM)] * 5,
        out_specs=pl.BlockSpec(memory_space=pltpu.VMEM),
        scratch_shapes=[
            pltpu.VMEM((4 * B, BLK, D_MODEL), jnp.bfloat16),
            pltpu.VMEM((4 * B, BLK, D_MODEL), jnp.bfloat16),
            pltpu.SemaphoreType.DMA((4 * B,)),
            pltpu.SemaphoreType.DMA((4 * B,)),
        ],
        compiler_params=pltpu.CompilerParams(collective_id=0),
    )(x, Wq, k_loc, v_loc, Wo)
